# baseline (speedup 1.0000x reference)
import functools

import jax
import jax.numpy as jnp
from jax.experimental import pallas as pl
from jax.experimental.pallas import tpu as pltpu

_B = 4096
_M = 4096
_S = 64


def _bdot(a, b):
    return jnp.dot(a.astype(jnp.bfloat16), b.astype(jnp.bfloat16),
                   preferred_element_type=jnp.float32)


def _fused_kernel(x_ref, ids_ref, wcat_ref, bcat_ref,
                  w1_ref, b1_ref, g1_ref, be1_ref,
                  w2_ref, b2_ref, g2_ref, be2_ref,
                  w3_ref, b3_ref, g3_ref, be3_ref,
                  wo_ref, bo_ref,
                  w_out_ref, pred_ref, mol_ref, *,
                  nblocks, B, S, T, D, H):
    k = pl.program_id(0)
    M = x_ref.shape[0]
    x = x_ref[...]
    ids = ids_ref[...].reshape(1, M)

    z = _bdot(x, wcat_ref[...])
    w = jax.nn.sigmoid(z + bcat_ref[...])
    w_out_ref[...] = w

    y = jnp.concatenate([x * w[:, t:t + 1] for t in range(T)], axis=1)
    y1 = y.astype(jnp.bfloat16)
    y2 = (y - y1.astype(jnp.float32)).astype(jnp.bfloat16)

    @pl.when(k == 0)
    def _():
        mol_ref[...] = jnp.zeros_like(mol_ref)

    idsb = jnp.broadcast_to(ids, (S, M))

    def cond(s0):
        return s0 < B

    def body(s0):
        s0a = jnp.minimum((s0 // 8) * 8, B - S)
        rows = s0a + jax.lax.broadcasted_iota(jnp.int32, (S, M), 0)
        oh = ((rows == idsb) & (idsb >= s0)).astype(jnp.bfloat16)
        part = (jnp.dot(oh, y1, preferred_element_type=jnp.float32)
                + jnp.dot(oh, y2, preferred_element_type=jnp.float32))
        mol_ref[pl.ds(s0a, S), :] += part
        return jnp.min(jnp.where(ids >= s0a + S, ids, B))

    jax.lax.while_loop(cond, body, jnp.min(ids))

    @pl.when(k == nblocks - 1)
    def _():
        def bn(h, g, be):
            mu = jnp.mean(h, axis=0, keepdims=True)
            var = jnp.mean((h - mu) ** 2, axis=0, keepdims=True)
            return g * (h - mu) / jnp.sqrt(var + 1e-5) + be

        for t in range(T):
            h = bn(jax.nn.relu(_bdot(mol_ref[:, t * D:(t + 1) * D],
                                     w1_ref[t]) + b1_ref[t]),
                   g1_ref[t], be1_ref[t])
            h = bn(jax.nn.relu(_bdot(h, w2_ref[t]) + b2_ref[t]),
                   g2_ref[t], be2_ref[t])
            h = bn(jax.nn.relu(_bdot(h, w3_ref[t]) + b3_ref[t]),
                   g3_ref[t], be3_ref[t])
            pred_ref[t] = _bdot(h, wo_ref[t]) + bo_ref[t]


def _forward(node_feats, segment_ids, W_att, b_att, W1, b1, g1, be1,
             W2, b2, g2, be2, W3, b3, g3, be3, Wout, bout, *, B, M, S):
    N, D = node_feats.shape
    T = W_att.shape[0]
    H = W1.shape[2]
    nblocks = N // M

    wcat = jnp.zeros((D, 8), jnp.float32).at[:, :T].set(W_att[:, :, 0].T)
    bcat = jnp.zeros((1, 8), jnp.float32).at[0, :T].set(b_att[:, 0])
    ids3 = segment_ids.reshape(nblocks, 1, M)
    woP = jnp.zeros((T, H, 8), jnp.float32).at[:, :, :1].set(Wout)
    boP = jnp.zeros((T, 1, 8), jnp.float32).at[:, 0, :1].set(bout)

    cspec = lambda shape: pl.BlockSpec(shape, lambda k: (0,) * len(shape))
    r1 = lambda v: v.reshape(T, 1, H)

    w8, pred8 = pl.pallas_call(
        functools.partial(_fused_kernel, nblocks=nblocks, B=B, S=S, T=T,
                          D=D, H=H),
        grid=(nblocks,),
        in_specs=[
            pl.BlockSpec((M, D), lambda k: (k, 0)),
            pl.BlockSpec((1, 1, M), lambda k: (k, 0, 0)),
            cspec((D, 8)), cspec((1, 8)),
            cspec((T, D, H)), cspec((T, 1, H)), cspec((T, 1, H)),
            cspec((T, 1, H)),
            cspec((T, H, H)), cspec((T, 1, H)), cspec((T, 1, H)),
            cspec((T, 1, H)),
            cspec((T, H, H)), cspec((T, 1, H)), cspec((T, 1, H)),
            cspec((T, 1, H)),
            cspec((T, H, 8)), cspec((T, 1, 8)),
        ],
        out_specs=[
            pl.BlockSpec((M, 8), lambda k: (k, 0)),
            cspec((T, B, 8)),
        ],
        out_shape=[
            jax.ShapeDtypeStruct((N, 8), jnp.float32),
            jax.ShapeDtypeStruct((T, B, 8), jnp.float32),
        ],
        scratch_shapes=[pltpu.VMEM((B, T * D), jnp.float32)],
    )(node_feats, ids3, wcat, bcat,
      W1, r1(b1), r1(g1), r1(be1),
      W2, r1(b2), r1(g2), r1(be2),
      W3, r1(b3), r1(g3), r1(be3),
      woP, boP)

    prediction_all = pred8[:, :, 0].T
    atom_weight_list = w8[:, :T].T.reshape(T, N, 1)
    return prediction_all, atom_weight_list


def kernel(node_feats, segment_ids, W_att, b_att, W_sh, b_sh, W1, b1, g1, be1,
           W2, b2, g2, be2, W3, b3, g3, be3, Wout, bout):
    return _forward(node_feats, segment_ids, W_att, b_att, W1, b1, g1, be1,
                    W2, b2, g2, be2, W3, b3, g3, be3, Wout, bout,
                    B=_B, M=_M, S=_S)


# speedup vs baseline: 1.8535x; 1.8535x over previous
import functools

import jax
import jax.numpy as jnp
from jax.experimental import pallas as pl
from jax.experimental.pallas import tpu as pltpu

_B = 4096
_M = 2048
_S = 64


def _bdot(a, b):
    return jnp.dot(a.astype(jnp.bfloat16), b.astype(jnp.bfloat16),
                   preferred_element_type=jnp.float32)


def _fused_kernel(x_ref, ids_ref, wcat_ref, bcat_ref,
                  w1_ref, b1_ref, g1_ref, be1_ref,
                  w2_ref, b2_ref, g2_ref, be2_ref,
                  w3_ref, b3_ref, g3_ref, be3_ref,
                  wo_ref, bo_ref,
                  w_out_ref, pred_ref, mol_ref, *,
                  nblocks, B, S, T, D, H):
    k = pl.program_id(0)
    M = x_ref.shape[0]
    x = x_ref[...]
    ids = ids_ref[...].reshape(1, M)

    z = _bdot(x, wcat_ref[...]) + bcat_ref[...]
    wt = jax.nn.sigmoid(z.T)
    w_out_ref[...] = wt

    w1t = wt.astype(jnp.bfloat16)
    w2t = (wt - w1t.astype(jnp.float32)).astype(jnp.bfloat16)
    x1 = x.astype(jnp.bfloat16)
    x2 = (x - x1.astype(jnp.float32)).astype(jnp.bfloat16)

    @pl.when(k == 0)
    def _():
        mol_ref[...] = jnp.zeros_like(mol_ref)

    idsb = jnp.broadcast_to(ids, (S, M))

    def cond(s0):
        return s0 < B

    def body(s0):
        s0a = jnp.minimum((s0 // 8) * 8, B - S)
        rows = s0a + jax.lax.broadcasted_iota(jnp.int32, (S, M), 0)
        oh = ((rows == idsb) & (idsb >= s0)).astype(jnp.bfloat16)
        for t in range(T):
            a1 = oh * w1t[t:t + 1]
            a2 = oh * w2t[t:t + 1]
            part = (jnp.dot(a1, x1, preferred_element_type=jnp.float32)
                    + jnp.dot(a1, x2, preferred_element_type=jnp.float32)
                    + jnp.dot(a2, x1, preferred_element_type=jnp.float32))
            mol_ref[pl.ds(s0a, S), t * D:(t + 1) * D] += part
        return jnp.min(jnp.where(ids >= s0a + S, ids, B))

    jax.lax.while_loop(cond, body, jnp.min(ids))

    @pl.when(k == nblocks - 1)
    def _():
        def bn(h, g, be):
            mu = jnp.mean(h, axis=0, keepdims=True)
            var = jnp.mean((h - mu) ** 2, axis=0, keepdims=True)
            return g * (h - mu) / jnp.sqrt(var + 1e-5) + be

        for t in range(T):
            h = bn(jax.nn.relu(_bdot(mol_ref[:, t * D:(t + 1) * D],
                                     w1_ref[t]) + b1_ref[t]),
                   g1_ref[t], be1_ref[t])
            h = bn(jax.nn.relu(_bdot(h, w2_ref[t]) + b2_ref[t]),
                   g2_ref[t], be2_ref[t])
            h = bn(jax.nn.relu(_bdot(h, w3_ref[t]) + b3_ref[t]),
                   g3_ref[t], be3_ref[t])
            pred_ref[t] = _bdot(h, wo_ref[t]) + bo_ref[t]


def _forward(node_feats, segment_ids, W_att, b_att, W1, b1, g1, be1,
             W2, b2, g2, be2, W3, b3, g3, be3, Wout, bout, *, B, M, S):
    N, D = node_feats.shape
    T = W_att.shape[0]
    H = W1.shape[2]
    nblocks = N // M

    wcat = jnp.zeros((D, 8), jnp.float32).at[:, :T].set(W_att[:, :, 0].T)
    bcat = jnp.zeros((1, 8), jnp.float32).at[0, :T].set(b_att[:, 0])
    ids3 = segment_ids.reshape(nblocks, 1, M)
    woP = jnp.zeros((T, H, 8), jnp.float32).at[:, :, :1].set(Wout)
    boP = jnp.zeros((T, 1, 8), jnp.float32).at[:, 0, :1].set(bout)

    cspec = lambda shape: pl.BlockSpec(shape, lambda k: (0,) * len(shape))
    r1 = lambda v: v.reshape(T, 1, H)

    w8, pred8 = pl.pallas_call(
        functools.partial(_fused_kernel, nblocks=nblocks, B=B, S=S, T=T,
                          D=D, H=H),
        grid=(nblocks,),
        in_specs=[
            pl.BlockSpec((M, D), lambda k: (k, 0)),
            pl.BlockSpec((1, 1, M), lambda k: (k, 0, 0)),
            cspec((D, 8)), cspec((1, 8)),
            cspec((T, D, H)), cspec((T, 1, H)), cspec((T, 1, H)),
            cspec((T, 1, H)),
            cspec((T, H, H)), cspec((T, 1, H)), cspec((T, 1, H)),
            cspec((T, 1, H)),
            cspec((T, H, H)), cspec((T, 1, H)), cspec((T, 1, H)),
            cspec((T, 1, H)),
            cspec((T, H, 8)), cspec((T, 1, 8)),
        ],
        out_specs=[
            pl.BlockSpec((8, M), lambda k: (0, k)),
            cspec((T, B, 8)),
        ],
        out_shape=[
            jax.ShapeDtypeStruct((8, N), jnp.float32),
            jax.ShapeDtypeStruct((T, B, 8), jnp.float32),
        ],
        scratch_shapes=[pltpu.VMEM((B, T * D), jnp.float32)],
    )(node_feats, ids3, wcat, bcat,
      W1, r1(b1), r1(g1), r1(be1),
      W2, r1(b2), r1(g2), r1(be2),
      W3, r1(b3), r1(g3), r1(be3),
      woP, boP)

    prediction_all = pred8[:, :, 0].T
    atom_weight_list = w8[:T].reshape(T, N, 1)
    return prediction_all, atom_weight_list


def kernel(node_feats, segment_ids, W_att, b_att, W_sh, b_sh, W1, b1, g1, be1,
           W2, b2, g2, be2, W3, b3, g3, be3, Wout, bout):
    return _forward(node_feats, segment_ids, W_att, b_att, W1, b1, g1, be1,
                    W2, b2, g2, be2, W3, b3, g3, be3, Wout, bout,
                    B=_B, M=_M, S=_S)


# speedup vs baseline: 1.8738x; 1.0109x over previous
import functools

import jax
import jax.numpy as jnp
from jax.experimental import pallas as pl
from jax.experimental.pallas import tpu as pltpu

_B = 4096
_M = 2048
_S = 48


def _bdot(a, b):
    return jnp.dot(a.astype(jnp.bfloat16), b.astype(jnp.bfloat16),
                   preferred_element_type=jnp.float32)


def _fused_kernel(x_ref, ids_ref, wcat_ref, bcat_ref,
                  w1_ref, b1_ref, g1_ref, be1_ref,
                  w2_ref, b2_ref, g2_ref, be2_ref,
                  w3_ref, b3_ref, g3_ref, be3_ref,
                  wo_ref, bo_ref,
                  w_out_ref, pred_ref, mol_ref, *,
                  nblocks, B, S, T, D, H):
    k = pl.program_id(0)
    M = x_ref.shape[0]
    x = x_ref[...]
    ids = ids_ref[...].reshape(1, M)

    z = _bdot(x, wcat_ref[...]) + bcat_ref[...]
    wt = jax.nn.sigmoid(z.T)
    w_out_ref[...] = wt

    w1t = wt.astype(jnp.bfloat16)
    w2t = (wt - w1t.astype(jnp.float32)).astype(jnp.bfloat16)
    x1 = x.astype(jnp.bfloat16)
    x2 = (x - x1.astype(jnp.float32)).astype(jnp.bfloat16)

    @pl.when(k == 0)
    def _():
        mol_ref[...] = jnp.zeros_like(mol_ref)

    idsb = jnp.broadcast_to(ids, (S, M))

    def cond(s0):
        return s0 < B

    def body(s0):
        s0a = jnp.minimum((s0 // 8) * 8, B - S)
        rows = s0a + jax.lax.broadcasted_iota(jnp.int32, (S, M), 0)
        oh = ((rows == idsb) & (idsb >= s0)).astype(jnp.bfloat16)
        for t in range(T):
            a1 = oh * w1t[t:t + 1]
            a2 = oh * w2t[t:t + 1]
            part = (jnp.dot(a1, x1, preferred_element_type=jnp.float32)
                    + jnp.dot(a1, x2, preferred_element_type=jnp.float32)
                    + jnp.dot(a2, x1, preferred_element_type=jnp.float32))
            mol_ref[pl.ds(s0a, S), t * D:(t + 1) * D] += part
        return jnp.min(jnp.where(ids >= s0a + S, ids, B))

    jax.lax.while_loop(cond, body, jnp.min(ids))

    @pl.when(k == nblocks - 1)
    def _():
        def bn(h, g, be):
            mu = jnp.mean(h, axis=0, keepdims=True)
            var = jnp.mean((h - mu) ** 2, axis=0, keepdims=True)
            return g * (h - mu) / jnp.sqrt(var + 1e-5) + be

        for t in range(T):
            h = bn(jax.nn.relu(_bdot(mol_ref[:, t * D:(t + 1) * D],
                                     w1_ref[t]) + b1_ref[t]),
                   g1_ref[t], be1_ref[t])
            h = bn(jax.nn.relu(_bdot(h, w2_ref[t]) + b2_ref[t]),
                   g2_ref[t], be2_ref[t])
            h = bn(jax.nn.relu(_bdot(h, w3_ref[t]) + b3_ref[t]),
                   g3_ref[t], be3_ref[t])
            pred_ref[t] = _bdot(h, wo_ref[t]) + bo_ref[t]


def _forward(node_feats, segment_ids, W_att, b_att, W1, b1, g1, be1,
             W2, b2, g2, be2, W3, b3, g3, be3, Wout, bout, *, B, M, S):
    N, D = node_feats.shape
    T = W_att.shape[0]
    H = W1.shape[2]
    nblocks = N // M

    wcat = jnp.zeros((D, 8), jnp.float32).at[:, :T].set(W_att[:, :, 0].T)
    bcat = jnp.zeros((1, 8), jnp.float32).at[0, :T].set(b_att[:, 0])
    ids3 = segment_ids.reshape(nblocks, 1, M)
    woP = jnp.zeros((T, H, 8), jnp.float32).at[:, :, :1].set(Wout)
    boP = jnp.zeros((T, 1, 8), jnp.float32).at[:, 0, :1].set(bout)

    cspec = lambda shape: pl.BlockSpec(shape, lambda k: (0,) * len(shape))
    r1 = lambda v: v.reshape(T, 1, H)

    w8, pred8 = pl.pallas_call(
        functools.partial(_fused_kernel, nblocks=nblocks, B=B, S=S, T=T,
                          D=D, H=H),
        grid=(nblocks,),
        in_specs=[
            pl.BlockSpec((M, D), lambda k: (k, 0)),
            pl.BlockSpec((1, 1, M), lambda k: (k, 0, 0)),
            cspec((D, 8)), cspec((1, 8)),
            cspec((T, D, H)), cspec((T, 1, H)), cspec((T, 1, H)),
            cspec((T, 1, H)),
            cspec((T, H, H)), cspec((T, 1, H)), cspec((T, 1, H)),
            cspec((T, 1, H)),
            cspec((T, H, H)), cspec((T, 1, H)), cspec((T, 1, H)),
            cspec((T, 1, H)),
            cspec((T, H, 8)), cspec((T, 1, 8)),
        ],
        out_specs=[
            pl.BlockSpec((8, M), lambda k: (0, k)),
            cspec((T, B, 8)),
        ],
        out_shape=[
            jax.ShapeDtypeStruct((8, N), jnp.float32),
            jax.ShapeDtypeStruct((T, B, 8), jnp.float32),
        ],
        scratch_shapes=[pltpu.VMEM((B, T * D), jnp.float32)],
    )(node_feats, ids3, wcat, bcat,
      W1, r1(b1), r1(g1), r1(be1),
      W2, r1(b2), r1(g2), r1(be2),
      W3, r1(b3), r1(g3), r1(be3),
      woP, boP)

    prediction_all = pred8[:, :, 0].T
    atom_weight_list = w8[:T].reshape(T, N, 1)
    return prediction_all, atom_weight_list


def kernel(node_feats, segment_ids, W_att, b_att, W_sh, b_sh, W1, b1, g1, be1,
           W2, b2, g2, be2, W3, b3, g3, be3, Wout, bout):
    return _forward(node_feats, segment_ids, W_att, b_att, W1, b1, g1, be1,
                    W2, b2, g2, be2, W3, b3, g3, be3, Wout, bout,
                    B=_B, M=_M, S=_S)
